# baseline (speedup 1.0000x reference)
import functools

import jax
import jax.numpy as jnp
from jax import lax
from jax.experimental import pallas as pl
from jax.experimental.pallas import tpu as pltpu
from jax.experimental.pallas import tpu_sc as plsc

N = 100000
D = 128
B = 1024

_NC = 2
_NS = 16
_NW = _NC * _NS
_NPW = 3136
_NPAD = _NW * _NPW
_TAIL = N - _NPW
_L = 16


@functools.cache
def _make_sc_gather():
    mesh = plsc.VectorSubcoreMesh(core_axis_name="c", subcore_axis_name="s")

    @functools.partial(
        pl.kernel,
        out_type=jax.ShapeDtypeStruct((_NPAD,), jnp.float32),
        mesh=mesh,
        scratch_types=[
            pltpu.VMEM((B,), jnp.float32),
            pltpu.VMEM((_NPW,), jnp.int32),
            pltpu.VMEM((_NPW,), jnp.float32),
        ],
        compiler_params=pltpu.CompilerParams(needs_layout_passes=False),
    )
    def sc_gather(g_hbm, idx_hbm, out_hbm, g_v, idx_v, out_v):
        wid = lax.axis_index("s") * _NC + lax.axis_index("c")
        base = jnp.where(wid == _NW - 1, _TAIL, wid * _NPW)
        pltpu.sync_copy(g_hbm, g_v)
        pltpu.sync_copy(idx_hbm.at[pl.ds(base, _NPW)], idx_v)

        def body(i, carry):
            sl = pl.ds(i * _L, _L)
            out_v[sl] = plsc.load_gather(g_v, [idx_v[sl]])
            return carry

        lax.fori_loop(0, _NPW // _L, body, 0)
        pltpu.sync_copy(out_v, out_hbm.at[pl.ds(base, _NPW)])

    return sc_gather


_ROWS = 4096


def _film_body(g_ref, s_ref, t_ref, z_ref, o_ref):
    nsl = _ROWS // D
    g3 = jax.lax.broadcast_in_dim(g_ref[...], (nsl, D, D), (0, 1))
    z3 = z_ref[...].reshape(nsl, D, D)
    s3 = s_ref[...].reshape(1, 1, D)
    t3 = t_ref[...].reshape(1, 1, D)
    o_ref[...] = (z3 * (1.0 + g3 * s3) + g3 * t3).reshape(_ROWS, D)


def kernel(z, g_graph, batch, s, t):
    idx = batch.astype(jnp.int32)
    g_nodes = _make_sc_gather()(g_graph, idx)
    g3 = g_nodes.reshape(_NPAD // D, D)
    return pl.pallas_call(
        _film_body,
        out_shape=jax.ShapeDtypeStruct((N, D), jnp.float32),
        grid=(pl.cdiv(N, _ROWS),),
        in_specs=[
            pl.BlockSpec((_ROWS // D, D), lambda i: (i, 0)),
            pl.BlockSpec((1, D), lambda i: (0, 0)),
            pl.BlockSpec((1, D), lambda i: (0, 0)),
            pl.BlockSpec((_ROWS, D), lambda i: (i, 0)),
        ],
        out_specs=pl.BlockSpec((_ROWS, D), lambda i: (i, 0)),
    )(g3, s.reshape(1, D), t.reshape(1, D), z)


# speedup vs baseline: 1.1198x; 1.1198x over previous
import functools

import jax
import jax.numpy as jnp
from jax import lax
from jax.experimental import pallas as pl
from jax.experimental.pallas import tpu as pltpu
from jax.experimental.pallas import tpu_sc as plsc

N = 100000
D = 128
B = 1024

_NC = 2
_NS = 16
_NW = _NC * _NS
_NPW = 3136
_NPAD = _NW * _NPW
_TAIL = N - _NPW
_L = 16


@functools.cache
def _make_sc_gather():
    mesh = plsc.VectorSubcoreMesh(core_axis_name="c", subcore_axis_name="s")

    @functools.partial(
        pl.kernel,
        out_type=jax.ShapeDtypeStruct((_NPAD,), jnp.float32),
        mesh=mesh,
        scratch_types=[
            pltpu.VMEM((B,), jnp.float32),
            pltpu.VMEM((_NPW,), jnp.int32),
            pltpu.VMEM((_NPW,), jnp.float32),
        ],
        compiler_params=pltpu.CompilerParams(needs_layout_passes=False),
    )
    def sc_gather(g_hbm, idx_hbm, out_hbm, g_v, idx_v, out_v):
        wid = lax.axis_index("s") * _NC + lax.axis_index("c")
        base = jnp.where(wid == _NW - 1, _TAIL, wid * _NPW)
        pltpu.sync_copy(g_hbm, g_v)
        pltpu.sync_copy(idx_hbm.at[pl.ds(base, _NPW)], idx_v)

        def body(i, carry):
            sl = pl.ds(i * _L, _L)
            out_v[sl] = plsc.load_gather(g_v, [idx_v[sl]])
            return carry

        lax.fori_loop(0, _NPW // _L, body, 0)
        pltpu.sync_copy(out_v, out_hbm.at[pl.ds(base, _NPW)])

    return sc_gather


_ROWS = 24576


def _film_body(g_ref, s_ref, t_ref, z_ref, o_ref):
    nsl = _ROWS // D
    g3 = jax.lax.broadcast_in_dim(g_ref[...], (nsl, D, D), (0, 1))
    z3 = z_ref[...].reshape(nsl, D, D)
    s3 = s_ref[...].reshape(1, 1, D)
    t3 = t_ref[...].reshape(1, 1, D)
    o_ref[...] = (z3 * (1.0 + g3 * s3) + g3 * t3).reshape(_ROWS, D)


def kernel(z, g_graph, batch, s, t):
    idx = batch.astype(jnp.int32)
    g_nodes = _make_sc_gather()(g_graph, idx)
    g3 = g_nodes.reshape(_NPAD // D, D)
    return pl.pallas_call(
        _film_body,
        out_shape=jax.ShapeDtypeStruct((N, D), jnp.float32),
        grid=(pl.cdiv(N, _ROWS),),
        in_specs=[
            pl.BlockSpec((_ROWS // D, D), lambda i: (i, 0)),
            pl.BlockSpec((1, D), lambda i: (0, 0)),
            pl.BlockSpec((1, D), lambda i: (0, 0)),
            pl.BlockSpec((_ROWS, D), lambda i: (i, 0)),
        ],
        out_specs=pl.BlockSpec((_ROWS, D), lambda i: (i, 0)),
    )(g3, s.reshape(1, D), t.reshape(1, D), z)


# speedup vs baseline: 1.1376x; 1.0159x over previous
import functools

import jax
import jax.numpy as jnp
from jax import lax
from jax.experimental import pallas as pl
from jax.experimental.pallas import tpu as pltpu
from jax.experimental.pallas import tpu_sc as plsc

N = 100000
D = 128
B = 1024

_NC = 2
_NS = 16
_NW = _NC * _NS
_NPW = 3136
_NPAD = _NW * _NPW
_TAIL = N - _NPW
_L = 16


@functools.cache
def _make_sc_gather():
    mesh = plsc.VectorSubcoreMesh(core_axis_name="c", subcore_axis_name="s")

    @functools.partial(
        pl.kernel,
        out_type=jax.ShapeDtypeStruct((_NPAD,), jnp.float32),
        mesh=mesh,
        scratch_types=[
            pltpu.VMEM((B,), jnp.float32),
            pltpu.VMEM((_NPW,), jnp.int32),
            pltpu.VMEM((_NPW,), jnp.float32),
        ],
        compiler_params=pltpu.CompilerParams(needs_layout_passes=False),
    )
    def sc_gather(g_hbm, idx_hbm, out_hbm, g_v, idx_v, out_v):
        wid = lax.axis_index("s") * _NC + lax.axis_index("c")
        base = jnp.where(wid == _NW - 1, _TAIL, wid * _NPW)
        pltpu.sync_copy(g_hbm, g_v)
        pltpu.sync_copy(idx_hbm.at[pl.ds(base, _NPW)], idx_v)

        @plsc.parallel_loop(0, _NPW, _L, unroll=8)
        def body(i):
            sl = pl.ds(i, _L)
            out_v[sl] = plsc.load_gather(g_v, [idx_v[sl]])
        pltpu.sync_copy(out_v, out_hbm.at[pl.ds(base, _NPW)])

    return sc_gather


_ROWS = 24576


def _film_body(g_ref, s_ref, t_ref, z_ref, o_ref):
    nsl = _ROWS // D
    g3 = jax.lax.broadcast_in_dim(g_ref[...], (nsl, D, D), (0, 1))
    z3 = z_ref[...].reshape(nsl, D, D)
    s3 = s_ref[...].reshape(1, 1, D)
    t3 = t_ref[...].reshape(1, 1, D)
    o_ref[...] = (z3 * (1.0 + g3 * s3) + g3 * t3).reshape(_ROWS, D)


def kernel(z, g_graph, batch, s, t):
    idx = batch.astype(jnp.int32)
    g_nodes = _make_sc_gather()(g_graph, idx)
    g3 = g_nodes.reshape(_NPAD // D, D)
    return pl.pallas_call(
        _film_body,
        out_shape=jax.ShapeDtypeStruct((N, D), jnp.float32),
        grid=(pl.cdiv(N, _ROWS),),
        in_specs=[
            pl.BlockSpec((_ROWS // D, D), lambda i: (i, 0)),
            pl.BlockSpec((1, D), lambda i: (0, 0)),
            pl.BlockSpec((1, D), lambda i: (0, 0)),
            pl.BlockSpec((_ROWS, D), lambda i: (i, 0)),
        ],
        out_specs=pl.BlockSpec((_ROWS, D), lambda i: (i, 0)),
    )(g3, s.reshape(1, D), t.reshape(1, D), z)
